# baseline (speedup 1.0000x reference)
import functools

import jax
import jax.numpy as jnp
from jax import lax
from jax.experimental import pallas as pl
from jax.experimental.pallas import tpu as pltpu
from jax.experimental.pallas import tpu_sc as plsc

B, T, Cn, Cc, V, D = 1024, 50, 16, 26, 100000, 32
N = B * T
NC, NS = 2, 16
NW = NC * NS
TOK_W = N // NW
NPASS = 2
TOKP = TOK_W // NPASS
CH = 80
NCHUNK = TOKP // CH
NSTREAM = Cc * NCHUNK
NGRP = TOKP // 16


def _sc_embed_sum(tables_flat, xcat2):
    mesh = plsc.VectorSubcoreMesh(core_axis_name="c", subcore_axis_name="s")

    @functools.partial(
        pl.kernel,
        mesh=mesh,
        out_type=jax.ShapeDtypeStruct((N, D), jnp.float32),
        compiler_params=pltpu.CompilerParams(
            use_tc_tiling_on_sc=False, needs_layout_passes=False
        ),
        scratch_types=[
            pltpu.VMEM((TOKP, Cc), jnp.int32),
            pltpu.VMEM((NPASS, Cc, TOKP), jnp.int32),
            pltpu.VMEM((TOK_W, D), jnp.float32),
            pltpu.SemaphoreType.DMA,
        ],
    )
    def k(tab_hbm, idx_hbm, out_hbm, tm_v, fm_v, acc_v, sem):
        wid = lax.axis_index("s") * NC + lax.axis_index("c")
        base = wid * TOK_W

        zero = jnp.zeros((16,), jnp.float32)

        def zero_body(n, c2):
            acc_v[n, pl.ds(0, 16)] = zero
            acc_v[n, pl.ds(16, 16)] = zero
            return c2

        lax.fori_loop(0, TOK_W, zero_body, 0)

        lanes = lax.iota(jnp.int32, 16)
        row_pat = lanes

        for p in range(NPASS):
            pbase = base + p * TOKP
            pltpu.sync_copy(idx_hbm.at[pl.ds(pbase, TOKP)], tm_v)

            def tr_body(c, c2):
                col = jnp.full((16,), 0, jnp.int32) + c
                off = c * V

                def tr_inner(g, c3):
                    rows = row_pat + g * 16
                    vals = plsc.load_gather(tm_v, [rows, col])
                    fm_v[p, c, pl.ds(g * 16, 16)] = vals + off
                    return c3

                lax.fori_loop(0, NGRP, tr_inner, 0)
                return c2

            lax.fori_loop(0, Cc, tr_body, 0)

            def fire_body(s, c2):
                c = s // NCHUNK
                ci = s % NCHUNK
                pltpu.async_copy(
                    tab_hbm.at[fm_v.at[p, c, pl.ds(ci * CH, CH)]],
                    acc_v.at[pl.ds(p * TOKP + ci * CH, CH)],
                    sem,
                    add=True,
                )
                return c2

            lax.fori_loop(0, NSTREAM, fire_body, 0)

        def drain_body(s, c2):
            pltpu.make_async_copy(
                tab_hbm.at[fm_v.at[0, 0, pl.ds(0, CH)]],
                acc_v.at[pl.ds(0, CH)],
                sem,
            ).wait()
            return c2

        lax.fori_loop(0, NPASS * NSTREAM, drain_body, 0)

        pltpu.sync_copy(acc_v, out_hbm.at[pl.ds(base, TOK_W)])

    return k(tables_flat, xcat2)


def _tc_finish(x_num2, W_num, b_num2, cat_sum, gamma2, beta2):
    BLK = 2048

    def body(x_ref, w_ref, b_ref, s_ref, g_ref, bt_ref, o_ref):
        num = jnp.dot(x_ref[...], w_ref[...], preferred_element_type=jnp.float32)
        x = num + b_ref[...] + s_ref[...] * (1.0 / Cc)
        m = jnp.mean(x, axis=-1, keepdims=True)
        v = jnp.mean((x - m) ** 2, axis=-1, keepdims=True)
        o_ref[...] = (x - m) * lax.rsqrt(v + 1e-5) * g_ref[...] + bt_ref[...]

    return pl.pallas_call(
        body,
        grid=(N // BLK,),
        in_specs=[
            pl.BlockSpec((BLK, Cn), lambda i: (i, 0)),
            pl.BlockSpec((Cn, D), lambda i: (0, 0)),
            pl.BlockSpec((1, D), lambda i: (0, 0)),
            pl.BlockSpec((BLK, D), lambda i: (i, 0)),
            pl.BlockSpec((1, D), lambda i: (0, 0)),
            pl.BlockSpec((1, D), lambda i: (0, 0)),
        ],
        out_specs=pl.BlockSpec((BLK, D), lambda i: (i, 0)),
        out_shape=jax.ShapeDtypeStruct((N, D), jnp.float32),
    )(x_num2, W_num, b_num2, cat_sum, gamma2, beta2)


def kernel(x_num, x_cat, W_num, b_num, tables, gamma, beta):
    xcat2 = x_cat.astype(jnp.int32).reshape(N, Cc)
    tabf = tables.reshape(Cc * V, D)
    cat_sum = _sc_embed_sum(tabf, xcat2)
    out = _tc_finish(
        x_num.reshape(N, Cn),
        W_num,
        b_num.reshape(1, D),
        cat_sum,
        gamma.reshape(1, D),
        beta.reshape(1, D),
    )
    return out.reshape(B, T, D)


# speedup vs baseline: 1.1872x; 1.1872x over previous
import functools

import jax
import jax.numpy as jnp
from jax import lax
from jax.experimental import pallas as pl
from jax.experimental.pallas import tpu as pltpu
from jax.experimental.pallas import tpu_sc as plsc

B, T, Cn, Cc, V, D = 1024, 50, 16, 26, 100000, 32
N = B * T
CP = 28
NC, NS = 2, 16
NW = NC * NS
TOK_W = N // NW
NPASS = 2
TOKP = TOK_W // NPASS
CH = 80
NCHUNK = TOKP // CH
NSTREAM = Cc * NCHUNK
NGRP = TOKP // 16
VB = 512
NVB = -(-V // VB)
VOUT = NVB * VB


NJ = CP * D // 128


def _tc_relayout(t832):
    def body(t_ref, o_ref):
        o_ref[...] = t_ref[...].T

    return pl.pallas_call(
        body,
        grid=(NJ, NVB),
        in_specs=[pl.BlockSpec((128, VB), lambda j, i: (j, i))],
        out_specs=pl.BlockSpec((VB, 128), lambda j, i: (j * NVB + i, 0)),
        out_shape=jax.ShapeDtypeStruct((NJ * VOUT, 128), jnp.float32),
    )(t832)


def _sc_embed_sum(tables_flat, xcat2):
    mesh = plsc.VectorSubcoreMesh(core_axis_name="c", subcore_axis_name="s")

    @functools.partial(
        pl.kernel,
        mesh=mesh,
        out_type=jax.ShapeDtypeStruct((N, D), jnp.float32),
        compiler_params=pltpu.CompilerParams(
            use_tc_tiling_on_sc=False, needs_layout_passes=False
        ),
        scratch_types=[
            pltpu.VMEM((TOKP, Cc), jnp.int32),
            pltpu.VMEM((NPASS, Cc, TOKP), jnp.int32),
            pltpu.VMEM((TOK_W, D), jnp.float32),
            pltpu.SemaphoreType.DMA,
        ],
    )
    def k(tab_hbm, idx_hbm, out_hbm, tm_v, fm_v, acc_v, sem):
        wid = lax.axis_index("s") * NC + lax.axis_index("c")
        base = wid * TOK_W

        zero = jnp.zeros((16,), jnp.float32)

        def zero_body(n, c2):
            acc_v[n, pl.ds(0, 16)] = zero
            acc_v[n, pl.ds(16, 16)] = zero
            return c2

        lax.fori_loop(0, TOK_W, zero_body, 0)

        row_pat = lax.iota(jnp.int32, 16)

        for p in range(NPASS):
            pbase = base + p * TOKP
            pltpu.sync_copy(idx_hbm.at[pl.ds(pbase, TOKP)], tm_v)

            def tr_body(c, c2):
                col = jnp.full((16,), 0, jnp.int32) + c
                coff = (c // 4) * (4 * VOUT) + (c % 4)

                def tr_inner(g, c3):
                    rows = row_pat + g * 16
                    vals = plsc.load_gather(tm_v, [rows, col])
                    fm_v[p, c, pl.ds(g * 16, 16)] = vals * 4 + coff
                    return c3

                lax.fori_loop(0, NGRP, tr_inner, 0)
                return c2

            lax.fori_loop(0, Cc, tr_body, 0)

            def fire_body(s, c2):
                c = s // NCHUNK
                ci = s % NCHUNK
                pltpu.async_copy(
                    tab_hbm.at[fm_v.at[p, c, pl.ds(ci * CH, CH)]],
                    acc_v.at[pl.ds(p * TOKP + ci * CH, CH)],
                    sem,
                    add=True,
                )
                return c2

            lax.fori_loop(0, NSTREAM, fire_body, 0)

        def drain_body(s, c2):
            pltpu.make_async_copy(
                tab_hbm.at[fm_v.at[0, 0, pl.ds(0, CH)]],
                acc_v.at[pl.ds(0, CH)],
                sem,
            ).wait()
            return c2

        lax.fori_loop(0, NPASS * NSTREAM, drain_body, 0)

        pltpu.sync_copy(acc_v, out_hbm.at[pl.ds(base, TOK_W)])

    return k(tables_flat, xcat2)


def _tc_finish(x_num2, W_num, b_num2, cat_sum, gamma2, beta2):
    BLK = 2048

    def body(x_ref, w_ref, b_ref, s_ref, g_ref, bt_ref, o_ref):
        num = jnp.dot(x_ref[...], w_ref[...], preferred_element_type=jnp.float32)
        x = num + b_ref[...] + s_ref[...] * (1.0 / Cc)
        m = jnp.mean(x, axis=-1, keepdims=True)
        v = jnp.mean((x - m) ** 2, axis=-1, keepdims=True)
        o_ref[...] = (x - m) * lax.rsqrt(v + 1e-5) * g_ref[...] + bt_ref[...]

    return pl.pallas_call(
        body,
        grid=(N // BLK,),
        in_specs=[
            pl.BlockSpec((BLK, Cn), lambda i: (i, 0)),
            pl.BlockSpec((Cn, D), lambda i: (0, 0)),
            pl.BlockSpec((1, D), lambda i: (0, 0)),
            pl.BlockSpec((BLK, D), lambda i: (i, 0)),
            pl.BlockSpec((1, D), lambda i: (0, 0)),
            pl.BlockSpec((1, D), lambda i: (0, 0)),
        ],
        out_specs=pl.BlockSpec((BLK, D), lambda i: (i, 0)),
        out_shape=jax.ShapeDtypeStruct((N, D), jnp.float32),
    )(x_num2, W_num, b_num2, cat_sum, gamma2, beta2)


def kernel(x_num, x_cat, W_num, b_num, tables, gamma, beta):
    xcat2 = x_cat.astype(jnp.int32).reshape(N, Cc)
    t832 = jnp.transpose(tables, (0, 2, 1)).reshape(Cc * D, V)
    tabf = _tc_relayout(t832).reshape(4 * NJ * VOUT, D)
    cat_sum = _sc_embed_sum(tabf, xcat2)
    out = _tc_finish(
        x_num.reshape(N, Cn),
        W_num,
        b_num.reshape(1, D),
        cat_sum,
        gamma.reshape(1, D),
        beta.reshape(1, D),
    )
    return out.reshape(B, T, D)


# speedup vs baseline: 2.1001x; 1.7690x over previous
import functools

import jax
import jax.numpy as jnp
from jax import lax
from jax.experimental import pallas as pl
from jax.experimental.pallas import tpu as pltpu
from jax.experimental.pallas import tpu_sc as plsc

B, T, Cn, Cc, V, D = 1024, 50, 16, 26, 100000, 32
N = B * T
CP = 28
NC, NS = 2, 16
NW = NC * NS
TOK_W = N // NW
NPASS = 2
TOKP = TOK_W // NPASS
CH = 80
NCHUNK = TOKP // CH
NSTREAM = Cc * NCHUNK
NGRP = TOKP // 16
VB = 2048
NVB = -(-V // VB)
VOUT = NVB * VB


NJ = CP * D // 128


def _tc_relayout(t832):
    def body(t_ref, o_ref):
        o_ref[...] = t_ref[...].T

    return pl.pallas_call(
        body,
        grid=(NJ, NVB),
        in_specs=[pl.BlockSpec((128, VB), lambda j, i: (j, i))],
        out_specs=pl.BlockSpec((VB, 128), lambda j, i: (j * NVB + i, 0)),
        out_shape=jax.ShapeDtypeStruct((NJ * VOUT, 128), jnp.float32),
    )(t832)


def _sc_embed_sum(tables_flat, xcat2):
    mesh = plsc.VectorSubcoreMesh(core_axis_name="c", subcore_axis_name="s")

    @functools.partial(
        pl.kernel,
        mesh=mesh,
        out_type=jax.ShapeDtypeStruct((N, D), jnp.float32),
        compiler_params=pltpu.CompilerParams(
            use_tc_tiling_on_sc=False, needs_layout_passes=False
        ),
        scratch_types=[
            pltpu.VMEM((TOKP, Cc), jnp.int32),
            pltpu.VMEM((NPASS, Cc, TOKP), jnp.int32),
            pltpu.VMEM((TOK_W, D), jnp.float32),
            pltpu.SemaphoreType.DMA,
        ],
    )
    def k(tab_hbm, idx_hbm, out_hbm, tm_v, fm_v, acc_v, sem):
        wid = lax.axis_index("s") * NC + lax.axis_index("c")
        base = wid * TOK_W

        zero = jnp.zeros((16,), jnp.float32)

        def zero_body(n, c2):
            acc_v[n, pl.ds(0, 16)] = zero
            acc_v[n, pl.ds(16, 16)] = zero
            return c2

        lax.fori_loop(0, TOK_W, zero_body, 0)

        row_pat = lax.iota(jnp.int32, 16)

        for p in range(NPASS):
            pbase = base + p * TOKP
            pltpu.sync_copy(idx_hbm.at[pl.ds(pbase, TOKP)], tm_v)

            def tr_body(c, c2):
                col = jnp.full((16,), 0, jnp.int32) + c
                coff = (c // 4) * (4 * VOUT) + (c % 4)

                def tr_inner(g, c3):
                    rows = row_pat + g * 16
                    vals = plsc.load_gather(tm_v, [rows, col])
                    fm_v[p, c, pl.ds(g * 16, 16)] = vals * 4 + coff
                    return c3

                lax.fori_loop(0, NGRP, tr_inner, 0)
                return c2

            lax.fori_loop(0, Cc, tr_body, 0)

            def fire_body(s, c2):
                c = s // NCHUNK
                ci = s % NCHUNK
                pltpu.async_copy(
                    tab_hbm.at[fm_v.at[p, c, pl.ds(ci * CH, CH)]],
                    acc_v.at[pl.ds(p * TOKP + ci * CH, CH)],
                    sem,
                    add=True,
                )
                return c2

            lax.fori_loop(0, NSTREAM, fire_body, 0)

        def drain_body(s, c2):
            pltpu.make_async_copy(
                tab_hbm.at[fm_v.at[0, 0, pl.ds(0, CH)]],
                acc_v.at[pl.ds(0, CH)],
                sem,
            ).wait()
            return c2

        lax.fori_loop(0, NPASS * NSTREAM, drain_body, 0)

        pltpu.sync_copy(acc_v, out_hbm.at[pl.ds(base, TOK_W)])

    return k(tables_flat, xcat2)


def _tc_finish(x_num2, W_num, b_num2, cat_sum, gamma2, beta2):
    BLK = 2048

    def body(x_ref, w_ref, b_ref, s_ref, g_ref, bt_ref, o_ref):
        num = jnp.dot(x_ref[...], w_ref[...], preferred_element_type=jnp.float32)
        x = num + b_ref[...] + s_ref[...] * (1.0 / Cc)
        m = jnp.mean(x, axis=-1, keepdims=True)
        v = jnp.mean((x - m) ** 2, axis=-1, keepdims=True)
        o_ref[...] = (x - m) * lax.rsqrt(v + 1e-5) * g_ref[...] + bt_ref[...]

    return pl.pallas_call(
        body,
        grid=(N // BLK,),
        in_specs=[
            pl.BlockSpec((BLK, Cn), lambda i: (i, 0)),
            pl.BlockSpec((Cn, D), lambda i: (0, 0)),
            pl.BlockSpec((1, D), lambda i: (0, 0)),
            pl.BlockSpec((BLK, D), lambda i: (i, 0)),
            pl.BlockSpec((1, D), lambda i: (0, 0)),
            pl.BlockSpec((1, D), lambda i: (0, 0)),
        ],
        out_specs=pl.BlockSpec((BLK, D), lambda i: (i, 0)),
        out_shape=jax.ShapeDtypeStruct((N, D), jnp.float32),
    )(x_num2, W_num, b_num2, cat_sum, gamma2, beta2)


def kernel(x_num, x_cat, W_num, b_num, tables, gamma, beta):
    xcat2 = x_cat.astype(jnp.int32).reshape(N, Cc)
    t832 = jnp.transpose(tables, (0, 2, 1)).reshape(Cc * D, V)
    tabf = _tc_relayout(t832).reshape(4 * NJ * VOUT, D)
    cat_sum = _sc_embed_sum(tabf, xcat2)
    out = _tc_finish(
        x_num.reshape(N, Cn),
        W_num,
        b_num.reshape(1, D),
        cat_sum,
        gamma.reshape(1, D),
        beta.reshape(1, D),
    )
    return out.reshape(B, T, D)


# speedup vs baseline: 2.4495x; 1.1664x over previous
import functools

import jax
import jax.numpy as jnp
from jax import lax
from jax.experimental import pallas as pl
from jax.experimental.pallas import tpu as pltpu
from jax.experimental.pallas import tpu_sc as plsc

B, T, Cn, Cc, V, D = 1024, 50, 16, 26, 100000, 32
N = B * T
CP = 28
NC, NS = 2, 16
NW = NC * NS
TOK_W = N // NW
NPASS = 2
TOKP = TOK_W // NPASS
CH = 80
NCHUNK = TOKP // CH
NSTREAM = Cc * NCHUNK
NGRP = TOKP // 16
VB = 4096
NVB = -(-V // VB)
VOUT = NVB * VB


NJ = CP * D // 128


def _tc_relayout(t832):
    def body(t_ref, o_ref):
        o_ref[...] = t_ref[...].T

    return pl.pallas_call(
        body,
        grid=(NJ, NVB),
        in_specs=[pl.BlockSpec((128, VB), lambda j, i: (j, i))],
        out_specs=pl.BlockSpec((VB, 128), lambda j, i: (j * NVB + i, 0)),
        out_shape=jax.ShapeDtypeStruct((NJ * VOUT, 128), jnp.float32),
    )(t832)


def _sc_embed_sum(tables_flat, xcat2):
    mesh = plsc.VectorSubcoreMesh(core_axis_name="c", subcore_axis_name="s")

    @functools.partial(
        pl.kernel,
        mesh=mesh,
        out_type=jax.ShapeDtypeStruct((N, D), jnp.float32),
        compiler_params=pltpu.CompilerParams(
            use_tc_tiling_on_sc=False, needs_layout_passes=False
        ),
        scratch_types=[
            pltpu.VMEM((TOKP, Cc), jnp.int32),
            pltpu.VMEM((NPASS, Cc, TOKP), jnp.int32),
            pltpu.VMEM((TOK_W, D), jnp.float32),
            pltpu.SemaphoreType.DMA,
        ],
    )
    def k(tab_hbm, idx_hbm, out_hbm, tm_v, fm_v, acc_v, sem):
        wid = lax.axis_index("s") * NC + lax.axis_index("c")
        base = wid * TOK_W

        zero = jnp.zeros((16,), jnp.float32)

        def zero_body(n, c2):
            acc_v[n, pl.ds(0, 16)] = zero
            acc_v[n, pl.ds(16, 16)] = zero
            return c2

        lax.fori_loop(0, TOK_W, zero_body, 0)

        row_pat = lax.iota(jnp.int32, 16)

        for p in range(NPASS):
            pbase = base + p * TOKP
            pltpu.sync_copy(idx_hbm.at[pl.ds(pbase, TOKP)], tm_v)

            def tr_body(c, c2):
                col = jnp.full((16,), 0, jnp.int32) + c
                coff = (c // 4) * (4 * VOUT) + (c % 4)

                def tr_inner(g, c3):
                    rows = row_pat + g * 16
                    vals = plsc.load_gather(tm_v, [rows, col])
                    fm_v[p, c, pl.ds(g * 16, 16)] = vals * 4 + coff
                    return c3

                lax.fori_loop(0, NGRP, tr_inner, 0)
                return c2

            lax.fori_loop(0, Cc, tr_body, 0)

            def fire_body(s, c2):
                c = s // NCHUNK
                ci = s % NCHUNK
                pltpu.async_copy(
                    tab_hbm.at[fm_v.at[p, c, pl.ds(ci * CH, CH)]],
                    acc_v.at[pl.ds(p * TOKP + ci * CH, CH)],
                    sem,
                    add=True,
                )
                return c2

            lax.fori_loop(0, NSTREAM, fire_body, 0)

        def drain_body(s, c2):
            pltpu.make_async_copy(
                tab_hbm.at[fm_v.at[0, 0, pl.ds(0, CH)]],
                acc_v.at[pl.ds(0, CH)],
                sem,
            ).wait()
            return c2

        lax.fori_loop(0, NPASS * NSTREAM, drain_body, 0)

        pltpu.sync_copy(acc_v, out_hbm.at[pl.ds(base, TOK_W)])

    return k(tables_flat, xcat2)


def _tc_finish(x_num2, W_num, b_num2, cat_sum, gamma2, beta2):
    BLK = 2048

    def body(x_ref, w_ref, b_ref, s_ref, g_ref, bt_ref, o_ref):
        num = jnp.dot(x_ref[...], w_ref[...], preferred_element_type=jnp.float32)
        x = num + b_ref[...] + s_ref[...] * (1.0 / Cc)
        m = jnp.mean(x, axis=-1, keepdims=True)
        v = jnp.mean((x - m) ** 2, axis=-1, keepdims=True)
        o_ref[...] = (x - m) * lax.rsqrt(v + 1e-5) * g_ref[...] + bt_ref[...]

    return pl.pallas_call(
        body,
        grid=(N // BLK,),
        in_specs=[
            pl.BlockSpec((BLK, Cn), lambda i: (i, 0)),
            pl.BlockSpec((Cn, D), lambda i: (0, 0)),
            pl.BlockSpec((1, D), lambda i: (0, 0)),
            pl.BlockSpec((BLK, D), lambda i: (i, 0)),
            pl.BlockSpec((1, D), lambda i: (0, 0)),
            pl.BlockSpec((1, D), lambda i: (0, 0)),
        ],
        out_specs=pl.BlockSpec((BLK, D), lambda i: (i, 0)),
        out_shape=jax.ShapeDtypeStruct((N, D), jnp.float32),
    )(x_num2, W_num, b_num2, cat_sum, gamma2, beta2)


def kernel(x_num, x_cat, W_num, b_num, tables, gamma, beta):
    xcat2 = x_cat.astype(jnp.int32).reshape(N, Cc)
    t832 = jnp.transpose(tables, (0, 2, 1)).reshape(Cc * D, V)
    tabf = _tc_relayout(t832).reshape(4 * NJ * VOUT, D)
    cat_sum = _sc_embed_sum(tabf, xcat2)
    out = _tc_finish(
        x_num.reshape(N, Cn),
        W_num,
        b_num.reshape(1, D),
        cat_sum,
        gamma.reshape(1, D),
        beta.reshape(1, D),
    )
    return out.reshape(B, T, D)


# speedup vs baseline: 2.6045x; 1.0633x over previous
import functools

import jax
import jax.numpy as jnp
from jax import lax
from jax.experimental import pallas as pl
from jax.experimental.pallas import tpu as pltpu
from jax.experimental.pallas import tpu_sc as plsc

B, T, Cn, Cc, V, D = 1024, 50, 16, 26, 100000, 32
N = B * T
CP = 28
NC, NS = 2, 16
NW = NC * NS
TOK_W = N // NW
NPASS = 2
TOKP = TOK_W // NPASS
CH = 80
NCHUNK = TOKP // CH
NSTREAM = Cc * NCHUNK
NGRP = TOKP // 16
VB = 8192
NVB = -(-V // VB)
VOUT = NVB * VB


NJ = CP * D // 128


def _tc_relayout(t832):
    def body(t_ref, o_ref):
        o_ref[...] = t_ref[...].T

    return pl.pallas_call(
        body,
        grid=(NJ, NVB),
        in_specs=[pl.BlockSpec((128, VB), lambda j, i: (j, i))],
        out_specs=pl.BlockSpec((VB, 128), lambda j, i: (j * NVB + i, 0)),
        out_shape=jax.ShapeDtypeStruct((NJ * VOUT, 128), jnp.float32),
    )(t832)


def _sc_embed_sum(tables_flat, xcat2):
    mesh = plsc.VectorSubcoreMesh(core_axis_name="c", subcore_axis_name="s")

    @functools.partial(
        pl.kernel,
        mesh=mesh,
        out_type=jax.ShapeDtypeStruct((N, D), jnp.float32),
        compiler_params=pltpu.CompilerParams(
            use_tc_tiling_on_sc=False, needs_layout_passes=False
        ),
        scratch_types=[
            pltpu.VMEM((TOKP, Cc), jnp.int32),
            pltpu.VMEM((NPASS, Cc, TOKP), jnp.int32),
            pltpu.VMEM((TOK_W, D), jnp.float32),
            pltpu.SemaphoreType.DMA,
        ],
    )
    def k(tab_hbm, idx_hbm, out_hbm, tm_v, fm_v, acc_v, sem):
        wid = lax.axis_index("s") * NC + lax.axis_index("c")
        base = wid * TOK_W

        zero = jnp.zeros((16,), jnp.float32)

        def zero_body(n, c2):
            acc_v[n, pl.ds(0, 16)] = zero
            acc_v[n, pl.ds(16, 16)] = zero
            return c2

        lax.fori_loop(0, TOK_W, zero_body, 0)

        row_pat = lax.iota(jnp.int32, 16)

        for p in range(NPASS):
            pbase = base + p * TOKP
            pltpu.sync_copy(idx_hbm.at[pl.ds(pbase, TOKP)], tm_v)

            def tr_body(c, c2):
                col = jnp.full((16,), 0, jnp.int32) + c
                coff = (c // 4) * (4 * VOUT) + (c % 4)

                def tr_inner(g, c3):
                    rows = row_pat + g * 16
                    vals = plsc.load_gather(tm_v, [rows, col])
                    fm_v[p, c, pl.ds(g * 16, 16)] = vals * 4 + coff
                    return c3

                lax.fori_loop(0, NGRP, tr_inner, 0)
                return c2

            lax.fori_loop(0, Cc, tr_body, 0)

            def fire_body(s, c2):
                c = s // NCHUNK
                ci = s % NCHUNK
                pltpu.async_copy(
                    tab_hbm.at[fm_v.at[p, c, pl.ds(ci * CH, CH)]],
                    acc_v.at[pl.ds(p * TOKP + ci * CH, CH)],
                    sem,
                    add=True,
                )
                return c2

            lax.fori_loop(0, NSTREAM, fire_body, 0)

        def drain_body(s, c2):
            pltpu.make_async_copy(
                tab_hbm.at[fm_v.at[0, 0, pl.ds(0, CH)]],
                acc_v.at[pl.ds(0, CH)],
                sem,
            ).wait()
            return c2

        lax.fori_loop(0, NPASS * NSTREAM, drain_body, 0)

        pltpu.sync_copy(acc_v, out_hbm.at[pl.ds(base, TOK_W)])

    return k(tables_flat, xcat2)


def _tc_finish(x_num2, W_num, b_num2, cat_sum, gamma2, beta2):
    BLK = 2048

    def body(x_ref, w_ref, b_ref, s_ref, g_ref, bt_ref, o_ref):
        num = jnp.dot(x_ref[...], w_ref[...], preferred_element_type=jnp.float32)
        x = num + b_ref[...] + s_ref[...] * (1.0 / Cc)
        m = jnp.mean(x, axis=-1, keepdims=True)
        v = jnp.mean((x - m) ** 2, axis=-1, keepdims=True)
        o_ref[...] = (x - m) * lax.rsqrt(v + 1e-5) * g_ref[...] + bt_ref[...]

    return pl.pallas_call(
        body,
        grid=(N // BLK,),
        in_specs=[
            pl.BlockSpec((BLK, Cn), lambda i: (i, 0)),
            pl.BlockSpec((Cn, D), lambda i: (0, 0)),
            pl.BlockSpec((1, D), lambda i: (0, 0)),
            pl.BlockSpec((BLK, D), lambda i: (i, 0)),
            pl.BlockSpec((1, D), lambda i: (0, 0)),
            pl.BlockSpec((1, D), lambda i: (0, 0)),
        ],
        out_specs=pl.BlockSpec((BLK, D), lambda i: (i, 0)),
        out_shape=jax.ShapeDtypeStruct((N, D), jnp.float32),
    )(x_num2, W_num, b_num2, cat_sum, gamma2, beta2)


def kernel(x_num, x_cat, W_num, b_num, tables, gamma, beta):
    xcat2 = x_cat.astype(jnp.int32).reshape(N, Cc)
    t832 = jnp.transpose(tables, (0, 2, 1)).reshape(Cc * D, V)
    tabf = _tc_relayout(t832).reshape(4 * NJ * VOUT, D)
    cat_sum = _sc_embed_sum(tabf, xcat2)
    out = _tc_finish(
        x_num.reshape(N, Cn),
        W_num,
        b_num.reshape(1, D),
        cat_sum,
        gamma.reshape(1, D),
        beta.reshape(1, D),
    )
    return out.reshape(B, T, D)


# speedup vs baseline: 2.6969x; 1.0355x over previous
import functools

import jax
import jax.numpy as jnp
from jax import lax
from jax.experimental import pallas as pl
from jax.experimental.pallas import tpu as pltpu
from jax.experimental.pallas import tpu_sc as plsc

B, T, Cn, Cc, V, D = 1024, 50, 16, 26, 100000, 32
N = B * T
CP = 28
NC, NS = 2, 16
NW = NC * NS
TOK_W = N // NW
NPASS = 2
TOKP = TOK_W // NPASS
CH = 80
NCHUNK = TOKP // CH
NSTREAM = Cc * NCHUNK
NGRP = TOKP // 16
VB = 14336
NVB = -(-V // VB)
VOUT = NVB * VB


NJ = CP * D // 128


def _tc_relayout(t832):
    def body(t_ref, o_ref):
        o_ref[...] = t_ref[...].T

    return pl.pallas_call(
        body,
        grid=(NJ, NVB),
        in_specs=[pl.BlockSpec((128, VB), lambda j, i: (j, i))],
        out_specs=pl.BlockSpec((VB, 128), lambda j, i: (j * NVB + i, 0)),
        out_shape=jax.ShapeDtypeStruct((NJ * VOUT, 128), jnp.float32),
    )(t832)


def _sc_embed_sum(tables_flat, xcat2):
    mesh = plsc.VectorSubcoreMesh(core_axis_name="c", subcore_axis_name="s")

    @functools.partial(
        pl.kernel,
        mesh=mesh,
        out_type=jax.ShapeDtypeStruct((N, D), jnp.float32),
        compiler_params=pltpu.CompilerParams(
            use_tc_tiling_on_sc=False, needs_layout_passes=False
        ),
        scratch_types=[
            pltpu.VMEM((TOKP, Cc), jnp.int32),
            pltpu.VMEM((NPASS, Cc, TOKP), jnp.int32),
            pltpu.VMEM((TOK_W, D), jnp.float32),
            pltpu.SemaphoreType.DMA,
        ],
    )
    def k(tab_hbm, idx_hbm, out_hbm, tm_v, fm_v, acc_v, sem):
        wid = lax.axis_index("s") * NC + lax.axis_index("c")
        base = wid * TOK_W

        zero = jnp.zeros((16,), jnp.float32)

        def zero_body(n, c2):
            acc_v[n, pl.ds(0, 16)] = zero
            acc_v[n, pl.ds(16, 16)] = zero
            return c2

        lax.fori_loop(0, TOK_W, zero_body, 0)

        row_pat = lax.iota(jnp.int32, 16)

        for p in range(NPASS):
            pbase = base + p * TOKP
            pltpu.sync_copy(idx_hbm.at[pl.ds(pbase, TOKP)], tm_v)

            def tr_body(c, c2):
                col = jnp.full((16,), 0, jnp.int32) + c
                coff = (c // 4) * (4 * VOUT) + (c % 4)

                def tr_inner(g, c3):
                    rows = row_pat + g * 16
                    vals = plsc.load_gather(tm_v, [rows, col])
                    fm_v[p, c, pl.ds(g * 16, 16)] = vals * 4 + coff
                    return c3

                lax.fori_loop(0, NGRP, tr_inner, 0)
                return c2

            lax.fori_loop(0, Cc, tr_body, 0)

            def fire_body(s, c2):
                c = s // NCHUNK
                ci = s % NCHUNK
                pltpu.async_copy(
                    tab_hbm.at[fm_v.at[p, c, pl.ds(ci * CH, CH)]],
                    acc_v.at[pl.ds(p * TOKP + ci * CH, CH)],
                    sem,
                    add=True,
                )
                return c2

            lax.fori_loop(0, NSTREAM, fire_body, 0)

        def drain_body(s, c2):
            pltpu.make_async_copy(
                tab_hbm.at[fm_v.at[0, 0, pl.ds(0, CH)]],
                acc_v.at[pl.ds(0, CH)],
                sem,
            ).wait()
            return c2

        lax.fori_loop(0, NPASS * NSTREAM, drain_body, 0)

        pltpu.sync_copy(acc_v, out_hbm.at[pl.ds(base, TOK_W)])

    return k(tables_flat, xcat2)


def _tc_finish(x_num2, W_num, b_num2, cat_sum, gamma2, beta2):
    BLK = 2048

    def body(x_ref, w_ref, b_ref, s_ref, g_ref, bt_ref, o_ref):
        num = jnp.dot(x_ref[...], w_ref[...], preferred_element_type=jnp.float32)
        x = num + b_ref[...] + s_ref[...] * (1.0 / Cc)
        m = jnp.mean(x, axis=-1, keepdims=True)
        v = jnp.mean((x - m) ** 2, axis=-1, keepdims=True)
        o_ref[...] = (x - m) * lax.rsqrt(v + 1e-5) * g_ref[...] + bt_ref[...]

    return pl.pallas_call(
        body,
        grid=(N // BLK,),
        in_specs=[
            pl.BlockSpec((BLK, Cn), lambda i: (i, 0)),
            pl.BlockSpec((Cn, D), lambda i: (0, 0)),
            pl.BlockSpec((1, D), lambda i: (0, 0)),
            pl.BlockSpec((BLK, D), lambda i: (i, 0)),
            pl.BlockSpec((1, D), lambda i: (0, 0)),
            pl.BlockSpec((1, D), lambda i: (0, 0)),
        ],
        out_specs=pl.BlockSpec((BLK, D), lambda i: (i, 0)),
        out_shape=jax.ShapeDtypeStruct((N, D), jnp.float32),
    )(x_num2, W_num, b_num2, cat_sum, gamma2, beta2)


def kernel(x_num, x_cat, W_num, b_num, tables, gamma, beta):
    xcat2 = x_cat.astype(jnp.int32).reshape(N, Cc)
    t832 = jnp.transpose(tables, (0, 2, 1)).reshape(Cc * D, V)
    tabf = _tc_relayout(t832).reshape(4 * NJ * VOUT, D)
    cat_sum = _sc_embed_sum(tabf, xcat2)
    out = _tc_finish(
        x_num.reshape(N, Cn),
        W_num,
        b_num.reshape(1, D),
        cat_sum,
        gamma.reshape(1, D),
        beta.reshape(1, D),
    )
    return out.reshape(B, T, D)
